# baseline (speedup 1.0000x reference)
import jax
import jax.numpy as jnp
from jax import lax
from jax.experimental import pallas as pl
from jax.experimental.pallas import tpu as pltpu
from jax.experimental.pallas import tpu_sc as plsc

EMB = 32
BATCH = 16384
NC = 2
NS = 16
NW = NC * NS
BPW = BATCH // NW

_MLP_BLOCK = 2048


def _sc_gather_body(gid_hbm, aid_hbm, rid_hbm, gt_hbm, at_hbm, rt_hbm,
                    go_hbm, ao_hbm, ro_hbm,
                    gi_v, ai_v, ri_v, gr_v, ar_v, rr_v,
                    sem_g, sem_a, sem_r):
    wid = lax.axis_index("s") * NC + lax.axis_index("c")
    base = wid * BPW
    pltpu.sync_copy(gid_hbm.at[pl.ds(base, BPW)], gi_v)
    pltpu.sync_copy(aid_hbm.at[pl.ds(base, BPW)], ai_v)
    pltpu.sync_copy(rid_hbm.at[pl.ds(base, BPW)], ri_v)
    cg = pltpu.async_copy(gt_hbm.at[gi_v], gr_v, sem_g)
    ca = pltpu.async_copy(at_hbm.at[ai_v], ar_v, sem_a)
    cr = pltpu.async_copy(rt_hbm.at[ri_v], rr_v, sem_r)
    cg.wait()
    pltpu.sync_copy(gr_v, go_hbm.at[pl.ds(base, BPW)])
    ca.wait()
    pltpu.sync_copy(ar_v, ao_hbm.at[pl.ds(base, BPW)])
    cr.wait()
    pltpu.sync_copy(rr_v, ro_hbm.at[pl.ds(base, BPW)])


_EMB_OUT = jax.ShapeDtypeStruct((BATCH, EMB), jnp.float32)

_sc_gather = pl.kernel(
    _sc_gather_body,
    out_type=[_EMB_OUT, _EMB_OUT, _EMB_OUT],
    mesh=plsc.VectorSubcoreMesh(core_axis_name="c", subcore_axis_name="s"),
    scratch_types=[
        pltpu.VMEM((BPW,), jnp.int32),
        pltpu.VMEM((BPW,), jnp.int32),
        pltpu.VMEM((BPW,), jnp.int32),
        pltpu.VMEM((BPW, EMB), jnp.float32),
        pltpu.VMEM((BPW, EMB), jnp.float32),
        pltpu.VMEM((BPW, EMB), jnp.float32),
        pltpu.SemaphoreType.DMA,
        pltpu.SemaphoreType.DMA,
        pltpu.SemaphoreType.DMA,
    ],
    compiler_params=pltpu.CompilerParams(use_tc_tiling_on_sc=False),
)


def _mlp_body(g_ref, a_ref, r_ref, w1g_ref, w1a_ref, w1r_ref, b1_ref,
              w2_ref, b2_ref, w3_ref, b3_ref, o_ref):
    h = (jnp.dot(g_ref[...], w1g_ref[...], preferred_element_type=jnp.float32)
         + jnp.dot(a_ref[...], w1a_ref[...], preferred_element_type=jnp.float32)
         + jnp.dot(r_ref[...], w1r_ref[...], preferred_element_type=jnp.float32)
         + b1_ref[...])
    h = jnp.maximum(h, 0.0)
    h = jnp.maximum(
        jnp.dot(h, w2_ref[...], preferred_element_type=jnp.float32) + b2_ref[...],
        0.0)
    o_ref[...] = (jnp.dot(h, w3_ref[...], preferred_element_type=jnp.float32)
                  + b3_ref[...])


def _mlp(g, a, r, w1g, w1a, w1r, b1, w2, b2, w3, b3):
    n_blocks = BATCH // _MLP_BLOCK
    emb_spec = pl.BlockSpec((_MLP_BLOCK, EMB), lambda i: (i, 0))
    whole = lambda arr: pl.BlockSpec(arr.shape, lambda i: (0,) * arr.ndim)
    return pl.pallas_call(
        _mlp_body,
        grid=(n_blocks,),
        in_specs=[emb_spec, emb_spec, emb_spec,
                  whole(w1g), whole(w1a), whole(w1r), whole(b1),
                  whole(w2), whole(b2), whole(w3), whole(b3)],
        out_specs=pl.BlockSpec((_MLP_BLOCK, EMB), lambda i: (i, 0)),
        out_shape=jax.ShapeDtypeStruct((BATCH, EMB), jnp.float32),
    )(g, a, r, w1g, w1a, w1r, b1, w2, b2, w3, b3)


def kernel(genre_id, author_id, artist_id,
           genre_table, author_table, artist_table,
           W1, b1, W2, b2, W3, b3):
    g, a, r = _sc_gather(genre_id, author_id, artist_id,
                         genre_table, author_table, artist_table)
    return _mlp(g, a, r,
                W1[:EMB], W1[EMB:2 * EMB], W1[2 * EMB:],
                b1.reshape(1, -1), W2, b2.reshape(1, -1),
                W3, b3.reshape(1, -1))
